# baseline (speedup 1.0000x reference)
import functools

import jax
import jax.numpy as jnp
from jax import lax
from jax.experimental import pallas as pl
from jax.experimental.pallas import tpu as pltpu
from jax.experimental.pallas import tpu_sc as plsc

H = 7
LANES = 16
ROWS = 16


@functools.lru_cache(maxsize=None)
def _make_gather(vocab: int, b: int, l: int):
    info = plsc.get_sparse_core_info()
    num_cores = info.num_cores
    num_subcores = info.num_subcores
    assert num_cores == 2
    bpw = b // num_subcores
    assert b % num_subcores == 0 and bpw % (2 * ROWS) == 0
    nch = bpw // ROWS
    ngrp = ROWS * l // LANES

    mesh = plsc.VectorSubcoreMesh(core_axis_name="c", subcore_axis_name="s")

    @functools.partial(
        pl.kernel,
        mesh=mesh,
        out_type=(
            jax.ShapeDtypeStruct((b, l), jnp.float32),
            jax.ShapeDtypeStruct((b, l), jnp.float32),
        ),
        compiler_params=pltpu.CompilerParams(needs_layout_passes=False),
        scratch_types=[
            pltpu.VMEM((vocab,), jnp.float32),
            pltpu.VMEM((ROWS, l), jnp.int32),
            pltpu.VMEM((ROWS, l), jnp.int32),
            pltpu.VMEM((ROWS, l), jnp.float32),
            pltpu.VMEM((ROWS, l), jnp.float32),
            pltpu.SemaphoreType.DMA,
            pltpu.SemaphoreType.DMA,
            pltpu.SemaphoreType.DMA,
            pltpu.SemaphoreType.DMA,
        ],
    )
    def gather(tf_hbm, ta_hbm, ids_hbm, outf_hbm, outa_hbm, table_v, ids_v0,
               ids_v1, stag_v0, stag_v1, si0, si1, so0, so1):
        cid = lax.axis_index("c")
        base = lax.axis_index("s") * bpw
        ids_bufs = (ids_v0, ids_v1)
        stag_bufs = (stag_v0, stag_v1)
        sin = (si0, si1)
        sout = (so0, so1)

        def ids_copy(row0, bb):
            return pltpu.make_async_copy(
                ids_hbm.at[pl.ds(row0, ROWS), :], ids_bufs[bb], sin[bb]
            )

        offs = [o * LANES for o in range(l // LANES)]
        if l % LANES:
            offs.append(l - LANES)

        def compute(bb):
            idsb = ids_bufs[bb]
            stgb = stag_bufs[bb]

            def rowf(r, rc):
                for off in offs:
                    id16 = idsb[r, pl.ds(off, LANES)]
                    vals = plsc.load_gather(table_v, [id16])
                    stgb[r, pl.ds(off, LANES)] = vals
                return rc

            lax.fori_loop(0, ROWS, rowf, 0, unroll=4)

        for bb in range(2):
            ids_copy(base + bb * ROWS, bb).start()

        for t_idx, (table_hbm, out_hbm) in enumerate(
            ((tf_hbm, outf_hbm), (ta_hbm, outa_hbm))
        ):

            @pl.when(cid == t_idx)
            def _(table_hbm=table_hbm, out_hbm=out_hbm):
                pltpu.sync_copy(table_hbm, table_v)

                def out_copy(row0, bb):
                    return pltpu.make_async_copy(
                        stag_bufs[bb], out_hbm.at[pl.ds(row0, ROWS), :], sout[bb]
                    )

                def outer(i, carry):
                    for bb in range(2):
                        c = 2 * i + bb
                        row0 = base + c * ROWS
                        ids_copy(row0, bb).wait()

                        @pl.when(i > 0)
                        def _():
                            out_copy(row0, bb).wait()

                        compute(bb)
                        out_copy(row0, bb).start()

                        @pl.when(c + 2 < nch)
                        def _():
                            ids_copy(row0 + 2 * ROWS, bb).start()

                    return carry

                lax.fori_loop(0, nch // 2, outer, 0)
                for bb in range(2):
                    out_copy(base + (nch - 2 + bb) * ROWS, bb).wait()

    return gather


@functools.lru_cache(maxsize=None)
def _make_expand_tc(b: int, l: int):
    BT = 1024
    nb = b // BT

    def body(sc_ref, f_ref, a_ref, of_ref, oa_ref):
        ft = jnp.transpose(f_ref[...], (1, 0))
        at = jnp.transpose(a_ref[...], (1, 0))
        for j in range(H):
            of_ref[j] = ft * sc_ref[0, j]
            oa_ref[j] = at * sc_ref[1, j]

    return pl.pallas_call(
        body,
        grid=(nb,),
        in_specs=[
            pl.BlockSpec(memory_space=pltpu.SMEM),
            pl.BlockSpec((BT, l), lambda i: (i, 0)),
            pl.BlockSpec((BT, l), lambda i: (i, 0)),
        ],
        out_specs=[
            pl.BlockSpec((H, l, BT), lambda i: (0, 0, i)),
            pl.BlockSpec((H, l, BT), lambda i: (0, 0, i)),
        ],
        out_shape=[
            jax.ShapeDtypeStruct((H, l, b), jnp.float32),
            jax.ShapeDtypeStruct((H, l, b), jnp.float32),
        ],
    )


def kernel(ids, frequencies, amplitudes, decay):
    B, L = ids.shape
    ids32 = ids.astype(jnp.int32)
    h = jnp.arange(1, H + 1, dtype=jnp.float32)
    pad = jnp.zeros((1,), jnp.float32)
    scales = jnp.stack(
        [jnp.concatenate([h, pad]), jnp.concatenate([1.0 / (h ** decay), pad])]
    )
    fv, av = _make_gather(frequencies.shape[0], B, L)(
        frequencies, amplitudes, ids32
    )
    of, oa = _make_expand_tc(B, L)(scales, fv, av)
    return jnp.transpose(of, (2, 1, 0)), jnp.transpose(oa, (2, 1, 0))


# speedup vs baseline: 1.1436x; 1.1436x over previous
import functools

import jax
import jax.numpy as jnp
from jax import lax
from jax.experimental import pallas as pl
from jax.experimental.pallas import tpu as pltpu
from jax.experimental.pallas import tpu_sc as plsc

H = 7
LANES = 16
ROWS = 16


@functools.lru_cache(maxsize=None)
def _make_gather(vocab: int, b: int, l: int):
    info = plsc.get_sparse_core_info()
    num_cores = info.num_cores
    num_subcores = info.num_subcores
    assert num_cores == 2
    bpw = b // num_subcores
    assert b % num_subcores == 0 and bpw % (2 * ROWS) == 0
    nch = bpw // ROWS
    ngrp = ROWS * l // LANES

    mesh = plsc.VectorSubcoreMesh(core_axis_name="c", subcore_axis_name="s")

    @functools.partial(
        pl.kernel,
        mesh=mesh,
        out_type=(
            jax.ShapeDtypeStruct((b, l), jnp.float32),
            jax.ShapeDtypeStruct((b, l), jnp.float32),
        ),
        compiler_params=pltpu.CompilerParams(needs_layout_passes=False),
        scratch_types=[
            pltpu.VMEM((vocab,), jnp.float32),
            pltpu.VMEM((ROWS, l), jnp.int32),
            pltpu.VMEM((ROWS, l), jnp.int32),
            pltpu.VMEM((ROWS, l), jnp.float32),
            pltpu.VMEM((ROWS, l), jnp.float32),
            pltpu.SemaphoreType.DMA,
            pltpu.SemaphoreType.DMA,
            pltpu.SemaphoreType.DMA,
            pltpu.SemaphoreType.DMA,
        ],
    )
    def gather(tf_hbm, ta_hbm, ids_hbm, outf_hbm, outa_hbm, table_v, ids_v0,
               ids_v1, stag_v0, stag_v1, si0, si1, so0, so1):
        cid = lax.axis_index("c")
        base = lax.axis_index("s") * bpw
        ids_bufs = (ids_v0, ids_v1)
        stag_bufs = (stag_v0, stag_v1)
        sin = (si0, si1)
        sout = (so0, so1)

        def ids_copy(row0, bb):
            return pltpu.make_async_copy(
                ids_hbm.at[pl.ds(row0, ROWS), :], ids_bufs[bb], sin[bb]
            )

        offs = [o * LANES for o in range(l // LANES)]
        if l % LANES:
            offs.append(l - LANES)

        def compute(bb):
            idsb = ids_bufs[bb]
            stgb = stag_bufs[bb]

            def rowf(r, rc):
                for off in offs:
                    id16 = idsb[r, pl.ds(off, LANES)]
                    vals = plsc.load_gather(table_v, [id16])
                    stgb[r, pl.ds(off, LANES)] = vals
                return rc

            lax.fori_loop(0, ROWS, rowf, 0, unroll=2)

        for bb in range(2):
            ids_copy(base + bb * ROWS, bb).start()

        for t_idx, (table_hbm, out_hbm) in enumerate(
            ((tf_hbm, outf_hbm), (ta_hbm, outa_hbm))
        ):

            @pl.when(cid == t_idx)
            def _(table_hbm=table_hbm, out_hbm=out_hbm):
                pltpu.sync_copy(table_hbm, table_v)

                def out_copy(row0, bb):
                    return pltpu.make_async_copy(
                        stag_bufs[bb], out_hbm.at[pl.ds(row0, ROWS), :], sout[bb]
                    )

                def outer(i, carry):
                    for bb in range(2):
                        c = 2 * i + bb
                        row0 = base + c * ROWS
                        ids_copy(row0, bb).wait()

                        @pl.when(i > 0)
                        def _():
                            out_copy(row0, bb).wait()

                        compute(bb)
                        out_copy(row0, bb).start()

                        @pl.when(c + 2 < nch)
                        def _():
                            ids_copy(row0 + 2 * ROWS, bb).start()

                    return carry

                lax.fori_loop(0, nch // 2, outer, 0)
                for bb in range(2):
                    out_copy(base + (nch - 2 + bb) * ROWS, bb).wait()

    return gather


@functools.lru_cache(maxsize=None)
def _make_expand_tc(b: int, l: int):
    BT = 1024
    nb = b // BT

    def body(sc_ref, f_ref, a_ref, of_ref, oa_ref):
        ft = jnp.transpose(f_ref[...], (1, 0))
        at = jnp.transpose(a_ref[...], (1, 0))
        for j in range(H):
            of_ref[j] = ft * sc_ref[0, j]
            oa_ref[j] = at * sc_ref[1, j]

    return pl.pallas_call(
        body,
        grid=(nb,),
        in_specs=[
            pl.BlockSpec(memory_space=pltpu.SMEM),
            pl.BlockSpec((BT, l), lambda i: (i, 0)),
            pl.BlockSpec((BT, l), lambda i: (i, 0)),
        ],
        out_specs=[
            pl.BlockSpec((H, l, BT), lambda i: (0, 0, i)),
            pl.BlockSpec((H, l, BT), lambda i: (0, 0, i)),
        ],
        out_shape=[
            jax.ShapeDtypeStruct((H, l, b), jnp.float32),
            jax.ShapeDtypeStruct((H, l, b), jnp.float32),
        ],
    )


def kernel(ids, frequencies, amplitudes, decay):
    B, L = ids.shape
    ids32 = ids.astype(jnp.int32)
    h = jnp.arange(1, H + 1, dtype=jnp.float32)
    pad = jnp.zeros((1,), jnp.float32)
    scales = jnp.stack(
        [jnp.concatenate([h, pad]), jnp.concatenate([1.0 / (h ** decay), pad])]
    )
    fv, av = _make_gather(frequencies.shape[0], B, L)(
        frequencies, amplitudes, ids32
    )
    of, oa = _make_expand_tc(B, L)(scales, fv, av)
    return jnp.transpose(of, (2, 1, 0)), jnp.transpose(oa, (2, 1, 0))


# speedup vs baseline: 1.1537x; 1.0088x over previous
import functools

import jax
import jax.numpy as jnp
from jax import lax
from jax.experimental import pallas as pl
from jax.experimental.pallas import tpu as pltpu
from jax.experimental.pallas import tpu_sc as plsc

H = 7
LANES = 16
ROWS = 16


@functools.lru_cache(maxsize=None)
def _make_gather(vocab: int, b: int, l: int):
    info = plsc.get_sparse_core_info()
    num_cores = info.num_cores
    num_subcores = info.num_subcores
    assert num_cores == 2
    bpw = b // num_subcores
    assert b % num_subcores == 0 and bpw % (2 * ROWS) == 0
    nch = bpw // ROWS
    ngrp = ROWS * l // LANES

    mesh = plsc.VectorSubcoreMesh(core_axis_name="c", subcore_axis_name="s")

    @functools.partial(
        pl.kernel,
        mesh=mesh,
        out_type=(
            jax.ShapeDtypeStruct((b, l), jnp.float32),
            jax.ShapeDtypeStruct((b, l), jnp.float32),
        ),
        compiler_params=pltpu.CompilerParams(needs_layout_passes=False),
        scratch_types=[
            pltpu.VMEM((vocab,), jnp.float32),
            pltpu.VMEM((ROWS, l), jnp.int32),
            pltpu.VMEM((ROWS, l), jnp.int32),
            pltpu.VMEM((ROWS, l), jnp.float32),
            pltpu.VMEM((ROWS, l), jnp.float32),
            pltpu.SemaphoreType.DMA,
            pltpu.SemaphoreType.DMA,
            pltpu.SemaphoreType.DMA,
            pltpu.SemaphoreType.DMA,
        ],
    )
    def gather(tf_hbm, ta_hbm, ids_hbm, outf_hbm, outa_hbm, table_v, ids_v0,
               ids_v1, stag_v0, stag_v1, si0, si1, so0, so1):
        cid = lax.axis_index("c")
        base = lax.axis_index("s") * bpw
        ids_bufs = (ids_v0, ids_v1)
        stag_bufs = (stag_v0, stag_v1)
        sin = (si0, si1)
        sout = (so0, so1)

        def ids_copy(row0, bb):
            return pltpu.make_async_copy(
                ids_hbm.at[pl.ds(row0, ROWS), :], ids_bufs[bb], sin[bb]
            )

        offs = [o * LANES for o in range(l // LANES)]
        if l % LANES:
            offs.append(l - LANES)

        def compute(bb):
            idsb = ids_bufs[bb]
            stgb = stag_bufs[bb]

            def rowf(r, rc):
                for off in offs:
                    id16 = idsb[r, pl.ds(off, LANES)]
                    vals = plsc.load_gather(table_v, [id16])
                    stgb[r, pl.ds(off, LANES)] = vals
                return rc

            lax.fori_loop(0, ROWS, rowf, 0, unroll=2)

        for bb in range(2):
            ids_copy(base + bb * ROWS, bb).start()

        for t_idx, (table_hbm, out_hbm) in enumerate(
            ((tf_hbm, outf_hbm), (ta_hbm, outa_hbm))
        ):

            @pl.when(cid == t_idx)
            def _(table_hbm=table_hbm, out_hbm=out_hbm):
                pltpu.sync_copy(table_hbm, table_v)

                def out_copy(row0, bb):
                    return pltpu.make_async_copy(
                        stag_bufs[bb], out_hbm.at[pl.ds(row0, ROWS), :], sout[bb]
                    )

                def outer(i, carry):
                    for bb in range(2):
                        c = 2 * i + bb
                        row0 = base + c * ROWS
                        ids_copy(row0, bb).wait()

                        @pl.when(i > 0)
                        def _():
                            out_copy(row0, bb).wait()

                        compute(bb)
                        out_copy(row0, bb).start()

                        @pl.when(c + 2 < nch)
                        def _():
                            ids_copy(row0 + 2 * ROWS, bb).start()

                    return carry

                lax.fori_loop(0, nch // 2, outer, 0)
                for bb in range(2):
                    out_copy(base + (nch - 2 + bb) * ROWS, bb).wait()

    return gather


@functools.lru_cache(maxsize=None)
def _make_expand_tc(b: int, l: int):
    BT = 2048
    nb = b // BT

    def body(sc_ref, f_ref, a_ref, of_ref, oa_ref):
        ft = jnp.transpose(f_ref[...], (1, 0))
        at = jnp.transpose(a_ref[...], (1, 0))
        for j in range(H):
            of_ref[j] = ft * sc_ref[0, j]
            oa_ref[j] = at * sc_ref[1, j]

    return pl.pallas_call(
        body,
        grid=(nb,),
        in_specs=[
            pl.BlockSpec(memory_space=pltpu.SMEM),
            pl.BlockSpec((BT, l), lambda i: (i, 0)),
            pl.BlockSpec((BT, l), lambda i: (i, 0)),
        ],
        out_specs=[
            pl.BlockSpec((H, l, BT), lambda i: (0, 0, i)),
            pl.BlockSpec((H, l, BT), lambda i: (0, 0, i)),
        ],
        out_shape=[
            jax.ShapeDtypeStruct((H, l, b), jnp.float32),
            jax.ShapeDtypeStruct((H, l, b), jnp.float32),
        ],
    )


def kernel(ids, frequencies, amplitudes, decay):
    B, L = ids.shape
    ids32 = ids.astype(jnp.int32)
    h = jnp.arange(1, H + 1, dtype=jnp.float32)
    pad = jnp.zeros((1,), jnp.float32)
    scales = jnp.stack(
        [jnp.concatenate([h, pad]), jnp.concatenate([1.0 / (h ** decay), pad])]
    )
    fv, av = _make_gather(frequencies.shape[0], B, L)(
        frequencies, amplitudes, ids32
    )
    of, oa = _make_expand_tc(B, L)(scales, fv, av)
    return jnp.transpose(of, (2, 1, 0)), jnp.transpose(oa, (2, 1, 0))
